# baseline (device time: 329101 ns/iter reference)
import functools

import jax
import jax.numpy as jnp
from jax import lax
from jax.experimental import pallas as pl
from jax.experimental.pallas import tpu as pltpu

N_DEV = 4
SQ = 2048
SKV_SHARD = 2048
SLIVER = 128
NEEDED = SKV_SHARD + SLIVER
HQ, DH = 8, 128
D = HQ * DH
QBLK = 512
WIN = 768
WINDOW = 128
SCALE = 0.08838834764831843


def kernel(x, Wq, K_ext, V_ext, Wo):
    x2 = x.reshape(SQ, D)
    K2 = K_ext.reshape(SKV_SHARD, D)
    V2 = V_ext.reshape(SKV_SHARD, D)

    def body(x_ref, wq_ref, k_ref, v_ref, wo_ref, out_ref,
             kbuf, vbuf, recv_sems, send_sems, local_sems):
        my = lax.axis_index("i")
        left = lax.rem(my + N_DEV - 1, N_DEV)
        right = lax.rem(my + 1, N_DEV)

        barrier = pltpu.get_barrier_semaphore()
        for nbr in (left, right):
            pl.semaphore_signal(barrier, inc=1, device_id=(nbr,),
                                device_id_type=pl.DeviceIdType.MESH)
        pl.semaphore_wait(barrier, 2)

        def rdma(src, dst, ssem, rsem, tgt):
            return pltpu.make_async_remote_copy(
                src_ref=src, dst_ref=dst, send_sem=ssem, recv_sem=rsem,
                device_id=(tgt,), device_id_type=pl.DeviceIdType.MESH)

        big = pl.ds(0, SKV_SHARD)
        sliv = pl.ds(SKV_SHARD, SLIVER)
        k_sliv_src = k_ref.at[pl.ds(0, SLIVER)]
        v_sliv_src = v_ref.at[pl.ds(0, SLIVER)]


        @pl.when(my == 0)
        def _():
            cpk = pltpu.make_async_copy(k_ref, kbuf.at[big], local_sems.at[0])
            cpv = pltpu.make_async_copy(v_ref, vbuf.at[big], local_sems.at[1])
            cpk.start()
            cpv.start()
            sends = [
                rdma(k_ref, kbuf.at[big], send_sems.at[0], recv_sems.at[0], 1),
                rdma(k_ref, kbuf.at[big], send_sems.at[1], recv_sems.at[0], 3),
                rdma(v_ref, vbuf.at[big], send_sems.at[2], recv_sems.at[1], 1),
                rdma(v_ref, vbuf.at[big], send_sems.at[3], recv_sems.at[1], 3),
            ]
            for s in sends:
                s.start()
            cpk.wait()
            cpv.wait()
            rk = rdma(kbuf.at[sliv], kbuf.at[sliv],
                      send_sems.at[4], recv_sems.at[2], 1)
            rv = rdma(vbuf.at[sliv], vbuf.at[sliv],
                      send_sems.at[5], recv_sems.at[3], 1)
            rk.wait_recv()
            rv.wait_recv()
            for s in sends:
                s.wait_send()

        @pl.when(my == 1)
        def _():
            cpk = pltpu.make_async_copy(k_sliv_src, kbuf.at[sliv],
                                        local_sems.at[0])
            cpv = pltpu.make_async_copy(v_sliv_src, vbuf.at[sliv],
                                        local_sems.at[1])
            cpk.start()
            cpv.start()
            sends = [
                rdma(k_sliv_src, kbuf.at[sliv],
                     send_sems.at[0], recv_sems.at[2], 0),
                rdma(k_sliv_src, kbuf.at[sliv],
                     send_sems.at[1], recv_sems.at[2], 2),
                rdma(v_sliv_src, vbuf.at[sliv],
                     send_sems.at[2], recv_sems.at[3], 0),
                rdma(v_sliv_src, vbuf.at[sliv],
                     send_sems.at[3], recv_sems.at[3], 2),
            ]
            for s in sends:
                s.start()
            fwd_k = rdma(kbuf.at[big], kbuf.at[big],
                         send_sems.at[4], recv_sems.at[0], 2)
            fwd_v = rdma(vbuf.at[big], vbuf.at[big],
                         send_sems.at[5], recv_sems.at[1], 2)
            fwd_k.wait_recv()
            fwd_k.start()
            fwd_v.wait_recv()
            fwd_v.start()
            cpk.wait()
            cpv.wait()
            for s in sends:
                s.wait_send()
            fwd_k.wait_send()
            fwd_v.wait_send()

        @pl.when(my == 2)
        def _():
            fwd_k = rdma(kbuf.at[sliv], kbuf.at[sliv],
                         send_sems.at[0], recv_sems.at[2], 3)
            fwd_v = rdma(vbuf.at[sliv], vbuf.at[sliv],
                         send_sems.at[1], recv_sems.at[3], 3)
            fwd_k.wait_recv()
            fwd_k.start()
            fwd_v.wait_recv()
            fwd_v.start()
            rk = rdma(kbuf.at[big], kbuf.at[big],
                      send_sems.at[2], recv_sems.at[0], 3)
            rv = rdma(vbuf.at[big], vbuf.at[big],
                      send_sems.at[3], recv_sems.at[1], 3)
            rk.wait_recv()
            rv.wait_recv()
            fwd_k.wait_send()
            fwd_v.wait_send()

        @pl.when(my == 3)
        def _():
            rbk = rdma(kbuf.at[big], kbuf.at[big],
                       send_sems.at[0], recv_sems.at[0], 0)
            rbv = rdma(vbuf.at[big], vbuf.at[big],
                       send_sems.at[1], recv_sems.at[1], 0)
            rsk = rdma(kbuf.at[sliv], kbuf.at[sliv],
                       send_sems.at[2], recv_sems.at[2], 2)
            rsv = rdma(vbuf.at[sliv], vbuf.at[sliv],
                       send_sems.at[3], recv_sems.at[3], 2)
            rbk.wait_recv()
            rbv.wait_recv()
            rsk.wait_recv()
            rsv.wait_recv()

        for i in range(SQ // QBLK):
            q0 = i * QBLK
            j0 = min(max(q0 - WINDOW, 0), NEEDED - WIN)
            xblk = x_ref[pl.ds(q0, QBLK), :]
            qblk = jnp.dot(xblk, wq_ref[...],
                           preferred_element_type=jnp.float32)
            qidx = q0 + lax.broadcasted_iota(jnp.int32, (QBLK, WIN), 0)
            kidx = j0 + lax.broadcasted_iota(jnp.int32, (QBLK, WIN), 1)
            mask = jnp.abs(qidx - kidx) <= WINDOW
            ctx_parts = []
            for h in range(HQ):
                qh = qblk[:, h * DH:(h + 1) * DH]
                kh = kbuf[pl.ds(j0, WIN), h * DH:(h + 1) * DH]
                vh = vbuf[pl.ds(j0, WIN), h * DH:(h + 1) * DH]
                s = lax.dot_general(
                    qh, kh, (((1,), (1,)), ((), ())),
                    preferred_element_type=jnp.float32) * SCALE
                s = jnp.where(mask, s, -1e9)
                m = jnp.max(s, axis=1, keepdims=True)
                w = jnp.exp(s - m)
                w = w / jnp.sum(w, axis=1, keepdims=True)
                ctx_parts.append(
                    jnp.dot(w, vh, preferred_element_type=jnp.float32))
            ctx = jnp.concatenate(ctx_parts, axis=1)
            out_ref[pl.ds(q0, QBLK), :] = jnp.dot(
                ctx, wo_ref[...], preferred_element_type=jnp.float32)

        @functools.partial(pl.run_scoped,
                           sem2=pltpu.SemaphoreType.REGULAR)
        def _(sem2):
            for nbr in (left, right):
                pl.semaphore_signal(sem2, inc=1, device_id=(nbr,),
                                    device_id_type=pl.DeviceIdType.MESH)
            pl.semaphore_wait(sem2, 2)

    out2 = pl.pallas_call(
        body,
        out_shape=jax.ShapeDtypeStruct((SQ, D), jnp.float32),
        in_specs=[
            pl.BlockSpec(memory_space=pltpu.VMEM),
            pl.BlockSpec(memory_space=pltpu.VMEM),
            pl.BlockSpec(memory_space=pltpu.MemorySpace.HBM),
            pl.BlockSpec(memory_space=pltpu.MemorySpace.HBM),
            pl.BlockSpec(memory_space=pltpu.VMEM),
        ],
        out_specs=pl.BlockSpec(memory_space=pltpu.VMEM),
        scratch_shapes=[
            pltpu.VMEM((NEEDED, D), jnp.float32),
            pltpu.VMEM((NEEDED, D), jnp.float32),
            pltpu.SemaphoreType.DMA((4,)),
            pltpu.SemaphoreType.DMA((6,)),
            pltpu.SemaphoreType.DMA((2,)),
        ],
        compiler_params=pltpu.CompilerParams(collective_id=0),
    )(x2, Wq, K2, V2, Wo)
    return out2.reshape(1, SQ, D)


# device time: 167690 ns/iter; 1.9626x vs baseline; 1.9626x over previous
import functools

import jax
import jax.numpy as jnp
from jax import lax
from jax.experimental import pallas as pl
from jax.experimental.pallas import tpu as pltpu

N_DEV = 4
SQ = 2048
SKV_SHARD = 2048
SLIVER = 128
NEEDED = SKV_SHARD + SLIVER
HQ, DH = 8, 128
D = HQ * DH
QBLK = 512
WIN = 768
WINDOW = 128
SCALE = 0.08838834764831843

CH_ROWS = 128
NCH = 16


def _dir_rows(c, base):
    return pl.ds(base + CH_ROWS * (c // 2), CH_ROWS)


def kernel(x, Wq, K_ext, V_ext, Wo):
    x2 = x.reshape(SQ, D)
    K2 = K_ext.reshape(SKV_SHARD, D)
    V2 = V_ext.reshape(SKV_SHARD, D)

    def body(x_ref, wq_ref, k_ref, v_ref, wo_ref, out_ref,
             kbuf, vbuf, cw_recv, ccw_recv, sliv_recv,
             cw_send, ccw_send, sliv_send, local_sems):
        my = lax.axis_index("i")
        left = lax.rem(my + N_DEV - 1, N_DEV)
        right = lax.rem(my + 1, N_DEV)

        barrier = pltpu.get_barrier_semaphore()
        for nbr in (left, right):
            pl.semaphore_signal(barrier, inc=1, device_id=(nbr,),
                                device_id_type=pl.DeviceIdType.MESH)
        pl.semaphore_wait(barrier, 2)

        def rdma(src, dst, ssem, rsem, tgt):
            return pltpu.make_async_remote_copy(
                src_ref=src, dst_ref=dst, send_sem=ssem, recv_sem=rsem,
                device_id=(tgt,), device_id_type=pl.DeviceIdType.MESH)

        def hbm_src(c, base):
            rows = _dir_rows(c, base)
            return (k_ref.at[rows] if c % 2 == 0 else v_ref.at[rows])

        def buf_at(c, base):
            rows = _dir_rows(c, base)
            return (kbuf.at[rows] if c % 2 == 0 else vbuf.at[rows])

        sliv = pl.ds(SKV_SHARD, SLIVER)
        k_sliv_src = k_ref.at[pl.ds(0, SLIVER)]
        v_sliv_src = v_ref.at[pl.ds(0, SLIVER)]

        @pl.when(my == 0)
        def _():
            cpk = pltpu.make_async_copy(k_ref, kbuf.at[pl.ds(0, SKV_SHARD)],
                                        local_sems.at[0])
            cpv = pltpu.make_async_copy(v_ref, vbuf.at[pl.ds(0, SKV_SHARD)],
                                        local_sems.at[1])
            cpk.start()
            cpv.start()
            sends = []
            for c in range(NCH):
                s = rdma(hbm_src(c, 0), buf_at(c, 0),
                         cw_send.at[c], cw_recv.at[c], 1)
                s.start()
                sends.append(s)
                s = rdma(hbm_src(c, 1024), buf_at(c, 1024),
                         ccw_send.at[c], ccw_recv.at[c], 3)
                s.start()
                sends.append(s)
            cpk.wait()
            cpv.wait()
            rk = rdma(kbuf.at[sliv], kbuf.at[sliv],
                      sliv_send.at[0], sliv_recv.at[0], 1)
            rv = rdma(vbuf.at[sliv], vbuf.at[sliv],
                      sliv_send.at[1], sliv_recv.at[1], 1)
            rk.wait_recv()
            rv.wait_recv()
            for s in sends:
                s.wait_send()

        @pl.when(my == 1)
        def _():
            cpk = pltpu.make_async_copy(k_sliv_src, kbuf.at[sliv],
                                        local_sems.at[0])
            cpv = pltpu.make_async_copy(v_sliv_src, vbuf.at[sliv],
                                        local_sems.at[1])
            cpk.start()
            cpv.start()
            sliv_sends = [
                rdma(k_sliv_src, kbuf.at[sliv],
                     sliv_send.at[0], sliv_recv.at[0], 0),
                rdma(v_sliv_src, vbuf.at[sliv],
                     sliv_send.at[1], sliv_recv.at[1], 0),
                rdma(k_sliv_src, kbuf.at[sliv],
                     sliv_send.at[2], sliv_recv.at[0], 2),
                rdma(v_sliv_src, vbuf.at[sliv],
                     sliv_send.at[3], sliv_recv.at[1], 2),
            ]
            for s in sliv_sends:
                s.start()
            fwds = []
            for c in range(NCH):
                d = rdma(buf_at(c, 0), buf_at(c, 0),
                         cw_send.at[c], cw_recv.at[c], 2)
                d.wait_recv()
                d.start()
                fwds.append(d)
            for c in range(NCH):
                d = rdma(buf_at(c, 1024), buf_at(c, 1024),
                         ccw_send.at[c], ccw_recv.at[c], 0)
                d.wait_recv()
            cpk.wait()
            cpv.wait()
            for s in sliv_sends:
                s.wait_send()
            for d in fwds:
                d.wait_send()

        @pl.when(my == 2)
        def _():
            sk = rdma(kbuf.at[sliv], kbuf.at[sliv],
                      sliv_send.at[0], sliv_recv.at[0], 3)
            sv = rdma(vbuf.at[sliv], vbuf.at[sliv],
                      sliv_send.at[1], sliv_recv.at[1], 3)
            sk.wait_recv()
            sk.start()
            sv.wait_recv()
            sv.start()
            fwds = []
            for c in range(NCH):
                d = rdma(buf_at(c, 0), buf_at(c, 0),
                         cw_send.at[c], cw_recv.at[c], 3)
                d.wait_recv()
                d.start()
                fwds.append(d)
                d = rdma(buf_at(c, 1024), buf_at(c, 1024),
                         ccw_send.at[c], ccw_recv.at[c], 1)
                d.wait_recv()
                d.start()
                fwds.append(d)
            sk.wait_send()
            sv.wait_send()
            for d in fwds:
                d.wait_send()

        @pl.when(my == 3)
        def _():
            fwds = []
            for c in range(NCH):
                d = rdma(buf_at(c, 1024), buf_at(c, 1024),
                         ccw_send.at[c], ccw_recv.at[c], 2)
                d.wait_recv()
                d.start()
                fwds.append(d)
            for c in range(NCH):
                d = rdma(buf_at(c, 0), buf_at(c, 0),
                         cw_send.at[c], cw_recv.at[c], 0)
                d.wait_recv()
            rsk = rdma(kbuf.at[sliv], kbuf.at[sliv],
                       sliv_send.at[0], sliv_recv.at[0], 2)
            rsv = rdma(vbuf.at[sliv], vbuf.at[sliv],
                       sliv_send.at[1], sliv_recv.at[1], 2)
            rsk.wait_recv()
            rsv.wait_recv()
            for d in fwds:
                d.wait_send()

        for i in range(SQ // QBLK):
            q0 = i * QBLK
            j0 = min(max(q0 - WINDOW, 0), NEEDED - WIN)
            xblk = x_ref[pl.ds(q0, QBLK), :]
            qblk = jnp.dot(xblk, wq_ref[...],
                           preferred_element_type=jnp.float32)
            qidx = q0 + lax.broadcasted_iota(jnp.int32, (QBLK, WIN), 0)
            kidx = j0 + lax.broadcasted_iota(jnp.int32, (QBLK, WIN), 1)
            mask = jnp.abs(qidx - kidx) <= WINDOW
            ctx_parts = []
            for h in range(HQ):
                qh = qblk[:, h * DH:(h + 1) * DH]
                kh = kbuf[pl.ds(j0, WIN), h * DH:(h + 1) * DH]
                vh = vbuf[pl.ds(j0, WIN), h * DH:(h + 1) * DH]
                s = lax.dot_general(
                    qh, kh, (((1,), (1,)), ((), ())),
                    preferred_element_type=jnp.float32) * SCALE
                s = jnp.where(mask, s, -1e9)
                m = jnp.max(s, axis=1, keepdims=True)
                w = jnp.exp(s - m)
                w = w / jnp.sum(w, axis=1, keepdims=True)
                ctx_parts.append(
                    jnp.dot(w, vh, preferred_element_type=jnp.float32))
            ctx = jnp.concatenate(ctx_parts, axis=1)
            out_ref[pl.ds(q0, QBLK), :] = jnp.dot(
                ctx, wo_ref[...], preferred_element_type=jnp.float32)

        @functools.partial(pl.run_scoped,
                           sem2=pltpu.SemaphoreType.REGULAR)
        def _(sem2):
            for nbr in (left, right):
                pl.semaphore_signal(sem2, inc=1, device_id=(nbr,),
                                    device_id_type=pl.DeviceIdType.MESH)
            pl.semaphore_wait(sem2, 2)

    out2 = pl.pallas_call(
        body,
        out_shape=jax.ShapeDtypeStruct((SQ, D), jnp.float32),
        in_specs=[
            pl.BlockSpec(memory_space=pltpu.VMEM),
            pl.BlockSpec(memory_space=pltpu.VMEM),
            pl.BlockSpec(memory_space=pltpu.MemorySpace.HBM),
            pl.BlockSpec(memory_space=pltpu.MemorySpace.HBM),
            pl.BlockSpec(memory_space=pltpu.VMEM),
        ],
        out_specs=pl.BlockSpec(memory_space=pltpu.VMEM),
        scratch_shapes=[
            pltpu.VMEM((NEEDED, D), jnp.float32),
            pltpu.VMEM((NEEDED, D), jnp.float32),
            pltpu.SemaphoreType.DMA((NCH,)),
            pltpu.SemaphoreType.DMA((NCH,)),
            pltpu.SemaphoreType.DMA((2,)),
            pltpu.SemaphoreType.DMA((NCH,)),
            pltpu.SemaphoreType.DMA((NCH,)),
            pltpu.SemaphoreType.DMA((4,)),
            pltpu.SemaphoreType.DMA((2,)),
        ],
        compiler_params=pltpu.CompilerParams(collective_id=0),
    )(x2, Wq, K2, V2, Wo)
    return out2.reshape(1, SQ, D)


# device time: 113751 ns/iter; 2.8932x vs baseline; 1.4742x over previous
import functools

import jax
import jax.numpy as jnp
from jax import lax
from jax.experimental import pallas as pl
from jax.experimental.pallas import tpu as pltpu

N_DEV = 4
SQ = 2048
SKV_SHARD = 2048
SLIVER = 128
HQ, DH = 8, 128
D = HQ * DH
QBLK = 512
NBLK = SQ // QBLK
WINDOW = 128
SCALE = 0.08838834764831843

CH_ROWS = 128
CH_PER_BLK = QBLK // CH_ROWS
KJ0 = (0, 384, 896, 1408)
KWID = (640, 768, 768, 640)
CW_BLOCKS = (0, 2)
CCW_BLOCKS = (1, 3)


def _chunk(b, k):
    return pl.ds(QBLK * b + CH_ROWS * k, CH_ROWS)


def kernel(x, Wq, K_ext, V_ext, Wo):
    x2 = x.reshape(SQ, D)
    K2 = K_ext.reshape(SKV_SHARD, D)
    V2 = V_ext.reshape(SKV_SHARD, D)

    def body(x_ref, wq_ref, k_ref, v_ref, wo_ref, out_ref,
             kwin, vwin, ctxbuf, slivctx, slivm, slivl,
             cw_recv, ccw_recv, sliv_recv,
             cw_send, ccw_send, sliv_send, local_sems):
        my = lax.axis_index("i")
        left = lax.rem(my + N_DEV - 1, N_DEV)
        right = lax.rem(my + 1, N_DEV)

        barrier = pltpu.get_barrier_semaphore()
        for nbr in (left, right):
            pl.semaphore_signal(barrier, inc=1, device_id=(nbr,),
                                device_id_type=pl.DeviceIdType.MESH)
        pl.semaphore_wait(barrier, 2)

        def rdma(src, dst, ssem, rsem, tgt):
            return pltpu.make_async_remote_copy(
                src_ref=src, dst_ref=dst, send_sem=ssem, recv_sem=rsem,
                device_id=(tgt,), device_id_type=pl.DeviceIdType.MESH)

        def blk_sems(b):
            if b in CW_BLOCKS:
                return cw_recv, cw_send, 4 * CW_BLOCKS.index(b)
            return ccw_recv, ccw_send, 4 * CCW_BLOCKS.index(b)

        def outproj(b):
            rows = pl.ds(QBLK * b, QBLK)
            out_ref[rows, :] = jnp.dot(ctxbuf[rows, :], wo_ref[...],
                                       preferred_element_type=jnp.float32)

        def fwd_chunk(b, k, tgt):
            recv_arr, send_arr, base = blk_sems(b)
            d = rdma(ctxbuf.at[_chunk(b, k)], ctxbuf.at[_chunk(b, k)],
                     send_arr.at[base + k], recv_arr.at[base + k], tgt)
            d.wait_recv()
            d.start()
            return d

        def wait_chunk(b, k):
            recv_arr, send_arr, base = blk_sems(b)
            d = rdma(ctxbuf.at[_chunk(b, k)], ctxbuf.at[_chunk(b, k)],
                     send_arr.at[base + k], recv_arr.at[base + k], 0)
            d.wait_recv()

        sliver_regions = (slivctx, slivm, slivl)

        def kv_copies(b):
            kj0, kw = KJ0[b], KWID[b]
            p = b % 2
            src_rows = pl.ds(kj0, kw)
            dst_rows = pl.ds(0, kw)
            return (
                pltpu.make_async_copy(k_ref.at[src_rows],
                                      kwin.at[p, dst_rows],
                                      local_sems.at[2 * p]),
                pltpu.make_async_copy(v_ref.at[src_rows],
                                      vwin.at[p, dst_rows],
                                      local_sems.at[2 * p + 1]),
            )

        @pl.when(my == 0)
        def _():
            for c in kv_copies(0):
                c.start()
            sends = []
            for b in range(NBLK):
                if b + 1 < NBLK:
                    for c in kv_copies(b + 1):
                        c.start()
                for c in kv_copies(b):
                    c.wait()
                q0 = QBLK * b
                kj0, kw = KJ0[b], KWID[b]
                p = b % 2
                qblk = jnp.dot(x_ref[pl.ds(q0, QBLK), :], wq_ref[...],
                               preferred_element_type=jnp.float32)
                qidx = q0 + lax.broadcasted_iota(jnp.int32, (QBLK, kw), 0)
                kidx = kj0 + lax.broadcasted_iota(jnp.int32, (QBLK, kw), 1)
                mask = jnp.abs(qidx - kidx) <= WINDOW
                if b == NBLK - 1:
                    for i, reg in enumerate(sliver_regions):
                        rdma(reg, reg, sliv_send.at[i], sliv_recv.at[i],
                             1).wait_recv()
                parts = []
                for h in range(HQ):
                    hs = slice(h * DH, (h + 1) * DH)
                    qh = qblk[:, hs]
                    s = lax.dot_general(
                        qh, kwin[p, pl.ds(0, kw), hs],
                        (((1,), (1,)), ((), ())),
                        preferred_element_type=jnp.float32) * SCALE
                    s = jnp.where(mask, s, -1e9)
                    m0 = jnp.max(s, axis=1, keepdims=True)
                    e = jnp.exp(s - m0)
                    l0 = jnp.sum(e, axis=1, keepdims=True)
                    c0 = jnp.dot(e, vwin[p, pl.ds(0, kw), hs],
                                 preferred_element_type=jnp.float32)
                    if b < NBLK - 1:
                        parts.append(c0 / l0)
                    else:
                        keep = QBLK - SLIVER
                        m1 = slivm[:, h:h + 1]
                        l1 = slivl[:, h:h + 1]
                        c1 = slivctx[:, hs]
                        mb = jnp.maximum(m0[keep:], m1)
                        a0 = jnp.exp(m0[keep:] - mb)
                        a1 = jnp.exp(m1 - mb)
                        denom = l0[keep:] * a0 + l1 * a1
                        bot = (c0[keep:] * a0 + c1 * a1) / denom
                        top = c0[:keep] / l0[:keep]
                        parts.append(jnp.concatenate([top, bot], axis=0))
                ctxbuf[pl.ds(q0, QBLK), :] = jnp.concatenate(parts, axis=1)
                recv_arr, send_arr, base = blk_sems(b)
                tgt = 1 if b in CW_BLOCKS else 3
                for k in range(CH_PER_BLK):
                    d = rdma(ctxbuf.at[_chunk(b, k)],
                             ctxbuf.at[_chunk(b, k)],
                             send_arr.at[base + k], recv_arr.at[base + k],
                             tgt)
                    d.start()
                    sends.append(d)
            for b in range(NBLK):
                outproj(b)
            for d in sends:
                d.wait_send()

        @pl.when(my == 1)
        def _():
            cpk = pltpu.make_async_copy(k_ref.at[pl.ds(0, SLIVER)],
                                        kwin.at[0, pl.ds(0, SLIVER)],
                                        local_sems.at[0])
            cpv = pltpu.make_async_copy(v_ref.at[pl.ds(0, SLIVER)],
                                        vwin.at[0, pl.ds(0, SLIVER)],
                                        local_sems.at[1])
            cpk.start()
            cpv.start()
            qs = jnp.dot(x_ref[pl.ds(SQ - SLIVER, SLIVER), :], wq_ref[...],
                         preferred_element_type=jnp.float32)
            qidx = (SQ - SLIVER) + lax.broadcasted_iota(
                jnp.int32, (SLIVER, SLIVER), 0)
            kidx = SKV_SHARD + lax.broadcasted_iota(
                jnp.int32, (SLIVER, SLIVER), 1)
            mask = jnp.abs(qidx - kidx) <= WINDOW
            cpk.wait()
            cpv.wait()
            c_parts, m_parts, l_parts = [], [], []
            for h in range(HQ):
                hs = slice(h * DH, (h + 1) * DH)
                s = lax.dot_general(
                    qs[:, hs], kwin[0, pl.ds(0, SLIVER), hs],
                    (((1,), (1,)), ((), ())),
                    preferred_element_type=jnp.float32) * SCALE
                s = jnp.where(mask, s, -1e9)
                m1 = jnp.max(s, axis=1, keepdims=True)
                e = jnp.exp(s - m1)
                l_parts.append(jnp.sum(e, axis=1, keepdims=True))
                m_parts.append(m1)
                c_parts.append(jnp.dot(e, vwin[0, pl.ds(0, SLIVER), hs],
                                       preferred_element_type=jnp.float32))
            slivctx[...] = jnp.concatenate(c_parts, axis=1)
            slivm[...] = jnp.concatenate(m_parts, axis=1)
            slivl[...] = jnp.concatenate(l_parts, axis=1)
            sliv_sends = []
            for i, reg in enumerate(sliver_regions):
                d = rdma(reg, reg, sliv_send.at[i], sliv_recv.at[i], 0)
                d.start()
                sliv_sends.append(d)
            fwds = []
            for b in CW_BLOCKS:
                for k in range(CH_PER_BLK):
                    fwds.append(fwd_chunk(b, k, 2))
            outproj(CW_BLOCKS[0])
            outproj(CW_BLOCKS[1])
            for b in CCW_BLOCKS:
                for k in range(CH_PER_BLK):
                    wait_chunk(b, k)
                outproj(b)
            for d in sliv_sends + fwds:
                d.wait_send()

        @pl.when(my == 2)
        def _():
            fwds = []
            for k in range(CH_PER_BLK):
                fwds.append(fwd_chunk(CW_BLOCKS[0], k, 3))
                fwds.append(fwd_chunk(CCW_BLOCKS[0], k, 1))
            outproj(CW_BLOCKS[0])
            for k in range(CH_PER_BLK):
                fwds.append(fwd_chunk(CW_BLOCKS[1], k, 3))
                fwds.append(fwd_chunk(CCW_BLOCKS[1], k, 1))
            outproj(CCW_BLOCKS[0])
            outproj(CW_BLOCKS[1])
            outproj(CCW_BLOCKS[1])
            for d in fwds:
                d.wait_send()

        @pl.when(my == 3)
        def _():
            fwds = []
            for b in CCW_BLOCKS:
                for k in range(CH_PER_BLK):
                    fwds.append(fwd_chunk(b, k, 2))
            outproj(CCW_BLOCKS[0])
            outproj(CCW_BLOCKS[1])
            for b in CW_BLOCKS:
                for k in range(CH_PER_BLK):
                    wait_chunk(b, k)
                outproj(b)
            for d in fwds:
                d.wait_send()

        @functools.partial(pl.run_scoped,
                           sem2=pltpu.SemaphoreType.REGULAR)
        def _(sem2):
            for nbr in (left, right):
                pl.semaphore_signal(sem2, inc=1, device_id=(nbr,),
                                    device_id_type=pl.DeviceIdType.MESH)
            pl.semaphore_wait(sem2, 2)

    out2 = pl.pallas_call(
        body,
        out_shape=jax.ShapeDtypeStruct((SQ, D), jnp.float32),
        in_specs=[
            pl.BlockSpec(memory_space=pltpu.VMEM),
            pl.BlockSpec(memory_space=pltpu.VMEM),
            pl.BlockSpec(memory_space=pltpu.MemorySpace.HBM),
            pl.BlockSpec(memory_space=pltpu.MemorySpace.HBM),
            pl.BlockSpec(memory_space=pltpu.VMEM),
        ],
        out_specs=pl.BlockSpec(memory_space=pltpu.VMEM),
        scratch_shapes=[
            pltpu.VMEM((2, 768, D), jnp.float32),
            pltpu.VMEM((2, 768, D), jnp.float32),
            pltpu.VMEM((SQ, D), jnp.float32),
            pltpu.VMEM((SLIVER, D), jnp.float32),
            pltpu.VMEM((SLIVER, HQ), jnp.float32),
            pltpu.VMEM((SLIVER, HQ), jnp.float32),
            pltpu.SemaphoreType.DMA((8,)),
            pltpu.SemaphoreType.DMA((8,)),
            pltpu.SemaphoreType.DMA((3,)),
            pltpu.SemaphoreType.DMA((8,)),
            pltpu.SemaphoreType.DMA((8,)),
            pltpu.SemaphoreType.DMA((3,)),
            pltpu.SemaphoreType.DMA((4,)),
        ],
        compiler_params=pltpu.CompilerParams(
            collective_id=0, vmem_limit_bytes=56 * 1024 * 1024),
    )(x2, Wq, K2, V2, Wo)
    return out2.reshape(1, SQ, D)


# device time: 102607 ns/iter; 3.2074x vs baseline; 1.1086x over previous
import functools

import jax
import jax.numpy as jnp
from jax import lax
from jax.experimental import pallas as pl
from jax.experimental.pallas import tpu as pltpu

N_DEV = 4
SQ = 2048
SKV_SHARD = 2048
SLIVER = 128
HQ, DH = 8, 128
D = HQ * DH
QBLK = 256
NBLK = SQ // QBLK
WINDOW = 128
SCALE = 0.08838834764831843

CH_ROWS = 128
CH_PER_BLK = QBLK // CH_ROWS
KJ0 = tuple(max(0, QBLK * b - WINDOW) for b in range(NBLK))
KWID = tuple(min(SKV_SHARD, QBLK * b + QBLK + WINDOW) - KJ0[b]
             for b in range(NBLK))
CW_BLOCKS = tuple(b for b in range(NBLK) if b % 2 == 0)
CCW_BLOCKS = tuple(b for b in range(NBLK) if b % 2 == 1)


def _chunk(b, k):
    return pl.ds(QBLK * b + CH_ROWS * k, CH_ROWS)


def kernel(x, Wq, K_ext, V_ext, Wo):
    x2 = x.reshape(SQ, D)
    K2 = K_ext.reshape(SKV_SHARD, D)
    V2 = V_ext.reshape(SKV_SHARD, D)

    def body(x_ref, wq_ref, k_ref, v_ref, wo_ref, out_ref,
             kwin, vwin, ctxbuf, slivctx, slivm, slivl,
             cw_recv, ccw_recv, sliv_recv,
             cw_send, ccw_send, sliv_send, local_sems):
        my = lax.axis_index("i")
        left = lax.rem(my + N_DEV - 1, N_DEV)
        right = lax.rem(my + 1, N_DEV)

        barrier = pltpu.get_barrier_semaphore()
        for nbr in (left, right):
            pl.semaphore_signal(barrier, inc=1, device_id=(nbr,),
                                device_id_type=pl.DeviceIdType.MESH)
        pl.semaphore_wait(barrier, 2)

        def rdma(src, dst, ssem, rsem, tgt):
            return pltpu.make_async_remote_copy(
                src_ref=src, dst_ref=dst, send_sem=ssem, recv_sem=rsem,
                device_id=(tgt,), device_id_type=pl.DeviceIdType.MESH)

        def blk_sems(b):
            if b in CW_BLOCKS:
                return cw_recv, cw_send, CH_PER_BLK * CW_BLOCKS.index(b)
            return ccw_recv, ccw_send, CH_PER_BLK * CCW_BLOCKS.index(b)

        def outproj(b):
            rows = pl.ds(QBLK * b, QBLK)
            out_ref[rows, :] = jnp.dot(ctxbuf[rows, :], wo_ref[...],
                                       preferred_element_type=jnp.float32)

        def outproj_chunk(b, k):
            rows = _chunk(b, k)
            out_ref[rows, :] = jnp.dot(ctxbuf[rows, :], wo_ref[...],
                                       preferred_element_type=jnp.float32)

        def fwd_chunk(b, k, tgt):
            recv_arr, send_arr, base = blk_sems(b)
            d = rdma(ctxbuf.at[_chunk(b, k)], ctxbuf.at[_chunk(b, k)],
                     send_arr.at[base + k], recv_arr.at[base + k], tgt)
            d.wait_recv()
            d.start()
            return d

        def wait_chunk(b, k):
            recv_arr, send_arr, base = blk_sems(b)
            d = rdma(ctxbuf.at[_chunk(b, k)], ctxbuf.at[_chunk(b, k)],
                     send_arr.at[base + k], recv_arr.at[base + k], 0)
            d.wait_recv()

        sliver_regions = (slivctx, slivm, slivl)

        def kv_copies(b):
            kj0, kw = KJ0[b], KWID[b]
            p = b % 2
            src_rows = pl.ds(kj0, kw)
            dst_rows = pl.ds(0, kw)
            return (
                pltpu.make_async_copy(k_ref.at[src_rows],
                                      kwin.at[p, dst_rows],
                                      local_sems.at[2 * p]),
                pltpu.make_async_copy(v_ref.at[src_rows],
                                      vwin.at[p, dst_rows],
                                      local_sems.at[2 * p + 1]),
            )

        @pl.when(my == 0)
        def _():
            for c in kv_copies(0):
                c.start()
            sends = []
            for b in range(NBLK):
                if b + 1 < NBLK:
                    for c in kv_copies(b + 1):
                        c.start()
                for c in kv_copies(b):
                    c.wait()
                q0 = QBLK * b
                kj0, kw = KJ0[b], KWID[b]
                p = b % 2
                qblk = jnp.dot(x_ref[pl.ds(q0, QBLK), :], wq_ref[...],
                               preferred_element_type=jnp.float32)
                qidx = q0 + lax.broadcasted_iota(jnp.int32, (QBLK, kw), 0)
                kidx = kj0 + lax.broadcasted_iota(jnp.int32, (QBLK, kw), 1)
                mask = jnp.abs(qidx - kidx) <= WINDOW
                if b == NBLK - 1:
                    for i, reg in enumerate(sliver_regions):
                        rdma(reg, reg, sliv_send.at[i], sliv_recv.at[i],
                             1).wait_recv()
                parts = []
                for h in range(HQ):
                    hs = slice(h * DH, (h + 1) * DH)
                    qh = qblk[:, hs]
                    s = lax.dot_general(
                        qh, kwin[p, pl.ds(0, kw), hs],
                        (((1,), (1,)), ((), ())),
                        preferred_element_type=jnp.float32) * SCALE
                    s = jnp.where(mask, s, -1e9)
                    m0 = jnp.max(s, axis=1, keepdims=True)
                    e = jnp.exp(s - m0)
                    l0 = jnp.sum(e, axis=1, keepdims=True)
                    c0 = jnp.dot(e, vwin[p, pl.ds(0, kw), hs],
                                 preferred_element_type=jnp.float32)
                    if b < NBLK - 1:
                        parts.append(c0 / l0)
                    else:
                        keep = QBLK - SLIVER
                        m1 = slivm[:, h:h + 1]
                        l1 = slivl[:, h:h + 1]
                        c1 = slivctx[:, hs]
                        mb = jnp.maximum(m0[keep:], m1)
                        a0 = jnp.exp(m0[keep:] - mb)
                        a1 = jnp.exp(m1 - mb)
                        denom = l0[keep:] * a0 + l1 * a1
                        bot = (c0[keep:] * a0 + c1 * a1) / denom
                        top = c0[:keep] / l0[:keep]
                        parts.append(jnp.concatenate([top, bot], axis=0))
                ctxbuf[pl.ds(q0, QBLK), :] = jnp.concatenate(parts, axis=1)
                recv_arr, send_arr, base = blk_sems(b)
                tgt = 1 if b in CW_BLOCKS else 3
                for k in range(CH_PER_BLK):
                    d = rdma(ctxbuf.at[_chunk(b, k)],
                             ctxbuf.at[_chunk(b, k)],
                             send_arr.at[base + k], recv_arr.at[base + k],
                             tgt)
                    d.start()
                    sends.append(d)
            for b in range(NBLK):
                outproj(b)
            for d in sends:
                d.wait_send()

        @pl.when(my == 1)
        def _():
            cpk = pltpu.make_async_copy(k_ref.at[pl.ds(0, SLIVER)],
                                        kwin.at[0, pl.ds(0, SLIVER)],
                                        local_sems.at[0])
            cpv = pltpu.make_async_copy(v_ref.at[pl.ds(0, SLIVER)],
                                        vwin.at[0, pl.ds(0, SLIVER)],
                                        local_sems.at[1])
            cpk.start()
            cpv.start()
            qs = jnp.dot(x_ref[pl.ds(SQ - SLIVER, SLIVER), :], wq_ref[...],
                         preferred_element_type=jnp.float32)
            qidx = (SQ - SLIVER) + lax.broadcasted_iota(
                jnp.int32, (SLIVER, SLIVER), 0)
            kidx = SKV_SHARD + lax.broadcasted_iota(
                jnp.int32, (SLIVER, SLIVER), 1)
            mask = jnp.abs(qidx - kidx) <= WINDOW
            cpk.wait()
            cpv.wait()
            c_parts, m_parts, l_parts = [], [], []
            for h in range(HQ):
                hs = slice(h * DH, (h + 1) * DH)
                s = lax.dot_general(
                    qs[:, hs], kwin[0, pl.ds(0, SLIVER), hs],
                    (((1,), (1,)), ((), ())),
                    preferred_element_type=jnp.float32) * SCALE
                s = jnp.where(mask, s, -1e9)
                m1 = jnp.max(s, axis=1, keepdims=True)
                e = jnp.exp(s - m1)
                l_parts.append(jnp.sum(e, axis=1, keepdims=True))
                m_parts.append(m1)
                c_parts.append(jnp.dot(e, vwin[0, pl.ds(0, SLIVER), hs],
                                       preferred_element_type=jnp.float32))
            slivctx[...] = jnp.concatenate(c_parts, axis=1)
            slivm[...] = jnp.concatenate(m_parts, axis=1)
            slivl[...] = jnp.concatenate(l_parts, axis=1)
            sliv_sends = []
            for i, reg in enumerate(sliver_regions):
                d = rdma(reg, reg, sliv_send.at[i], sliv_recv.at[i], 0)
                d.start()
                sliv_sends.append(d)
            fwds = []
            for b in CW_BLOCKS:
                for k in range(CH_PER_BLK):
                    fwds.append(fwd_chunk(b, k, 2))
                    outproj_chunk(b, k)
            for b in CCW_BLOCKS:
                for k in range(CH_PER_BLK):
                    wait_chunk(b, k)
                    outproj_chunk(b, k)
            for d in sliv_sends + fwds:
                d.wait_send()

        @pl.when(my == 2)
        def _():
            fwds = []
            for i in range(len(CW_BLOCKS)):
                for k in range(CH_PER_BLK):
                    fwds.append(fwd_chunk(CW_BLOCKS[i], k, 3))
                    outproj_chunk(CW_BLOCKS[i], k)
                    fwds.append(fwd_chunk(CCW_BLOCKS[i], k, 1))
                    outproj_chunk(CCW_BLOCKS[i], k)
            for d in fwds:
                d.wait_send()

        @pl.when(my == 3)
        def _():
            fwds = []
            for b in CCW_BLOCKS:
                for k in range(CH_PER_BLK):
                    fwds.append(fwd_chunk(b, k, 2))
                    outproj_chunk(b, k)
            for b in CW_BLOCKS:
                for k in range(CH_PER_BLK):
                    wait_chunk(b, k)
                    outproj_chunk(b, k)
            for d in fwds:
                d.wait_send()

        @functools.partial(pl.run_scoped,
                           sem2=pltpu.SemaphoreType.REGULAR)
        def _(sem2):
            for nbr in (left, right):
                pl.semaphore_signal(sem2, inc=1, device_id=(nbr,),
                                    device_id_type=pl.DeviceIdType.MESH)
            pl.semaphore_wait(sem2, 2)

    out2 = pl.pallas_call(
        body,
        out_shape=jax.ShapeDtypeStruct((SQ, D), jnp.float32),
        in_specs=[
            pl.BlockSpec(memory_space=pltpu.VMEM),
            pl.BlockSpec(memory_space=pltpu.VMEM),
            pl.BlockSpec(memory_space=pltpu.MemorySpace.HBM),
            pl.BlockSpec(memory_space=pltpu.MemorySpace.HBM),
            pl.BlockSpec(memory_space=pltpu.VMEM),
        ],
        out_specs=pl.BlockSpec(memory_space=pltpu.VMEM),
        scratch_shapes=[
            pltpu.VMEM((2, 512, D), jnp.float32),
            pltpu.VMEM((2, 512, D), jnp.float32),
            pltpu.VMEM((SQ, D), jnp.float32),
            pltpu.VMEM((SLIVER, D), jnp.float32),
            pltpu.VMEM((SLIVER, HQ), jnp.float32),
            pltpu.VMEM((SLIVER, HQ), jnp.float32),
            pltpu.SemaphoreType.DMA((8,)),
            pltpu.SemaphoreType.DMA((8,)),
            pltpu.SemaphoreType.DMA((3,)),
            pltpu.SemaphoreType.DMA((8,)),
            pltpu.SemaphoreType.DMA((8,)),
            pltpu.SemaphoreType.DMA((3,)),
            pltpu.SemaphoreType.DMA((4,)),
        ],
        compiler_params=pltpu.CompilerParams(
            collective_id=0, vmem_limit_bytes=56 * 1024 * 1024),
    )(x2, Wq, K2, V2, Wo)
    return out2.reshape(1, SQ, D)
